# Initial kernel scaffold; baseline (speedup 1.0000x reference)
import jax
import jax.numpy as jnp
from jax.experimental import pallas as pl


def kernel(Q, scores, W, b):
    raise NotImplementedError("write your pallas kernel here")


# speedup vs baseline: 1.4782x; 1.4782x over previous
import functools

import jax
import jax.numpy as jnp
from jax.experimental import pallas as pl

BATCH, SEQ, DIM = 2, 4096, 1024
BASE_K, MAX_K = 256, 512

ROWS_PER_BLOCK = 128
N = SEQ
LOGN = 12


def _topk_body(scores_ref, out_ref):
    x = scores_ref[...]
    lane = jax.lax.broadcasted_iota(jnp.int32, (1, N), 1)
    for p in range(1, LOGN + 1):
        desc = ((lane >> p) & 1) == 0
        for t in range(p - 1, -1, -1):
            s = 1 << t
            low = (lane & s) == 0
            take_max = low == desc
            partner = jnp.where(low, jnp.roll(x, -s, axis=1),
                                jnp.roll(x, s, axis=1))
            hi = jnp.maximum(x, partner)
            lo = jnp.minimum(x, partner)
            x = jnp.where(take_max, hi, lo)
    out_ref[...] = x[:, :MAX_K]


def _imp_body(q_ref, w_ref, b_ref, out_ref):
    z = jnp.sum(q_ref[...] * w_ref[...], axis=1, keepdims=True) + b_ref[0, 0]
    imp = jax.nn.sigmoid(z)
    out_ref[...] = (BASE_K + (MAX_K - BASE_K) * imp).astype(jnp.int32)


def kernel(Q, scores, W, b):
    rows = BATCH * SEQ
    scores_f = scores.reshape(rows, SEQ)
    q_f = Q.reshape(rows, DIM)

    vals = pl.pallas_call(
        _topk_body,
        grid=(rows // ROWS_PER_BLOCK,),
        in_specs=[pl.BlockSpec((ROWS_PER_BLOCK, SEQ), lambda i: (i, 0))],
        out_specs=pl.BlockSpec((ROWS_PER_BLOCK, MAX_K), lambda i: (i, 0)),
        out_shape=jax.ShapeDtypeStruct((rows, MAX_K), jnp.float32),
    )(scores_f)

    wb = W.reshape(1, DIM)
    bb = b.reshape(1, 1)
    k_out = pl.pallas_call(
        _imp_body,
        grid=(rows // 512,),
        in_specs=[
            pl.BlockSpec((512, DIM), lambda i: (i, 0)),
            pl.BlockSpec((1, DIM), lambda i: (0, 0)),
            pl.BlockSpec((1, 1), lambda i: (0, 0)),
        ],
        out_specs=pl.BlockSpec((512, 1), lambda i: (i, 0)),
        out_shape=jax.ShapeDtypeStruct((rows, 1), jnp.int32),
    )(q_f, wb, bb)

    return (vals.reshape(BATCH, SEQ, MAX_K), k_out.reshape(BATCH, SEQ))


# speedup vs baseline: 6.2625x; 4.2365x over previous
import jax
import jax.numpy as jnp
from jax.experimental import pallas as pl

BATCH, SEQ, DIM = 2, 4096, 1024
BASE_K, MAX_K = 256, 512

LANES = 128
N = SEQ


def _stage_slice(x, sp, dpsize):
    n, L = x.shape
    y = x.reshape(-1, 2, dpsize // (2 * sp), 2, sp, L)
    a = y[:, :, :, 0]
    b = y[:, :, :, 1]
    hi = jnp.maximum(a, b)
    lo = jnp.minimum(a, b)
    top = jnp.stack([hi[:, 0], lo[:, 1]], axis=1)
    bot = jnp.stack([lo[:, 0], hi[:, 1]], axis=1)
    z = jnp.stack([top, bot], axis=3)
    return z.reshape(n, L)


def _stage_mask(x, sp, m_takemax):
    n, L = x.shape
    y = x.reshape(-1, 2, sp, L)
    partner = jnp.stack([y[:, 1], y[:, 0]], axis=1).reshape(n, L)
    hi = jnp.maximum(x, partner)
    lo = jnp.minimum(x, partner)
    return jnp.where(m_takemax, hi, lo)


def _stage_roll(x, sp, m_low, m_takemax):
    partner = jnp.where(m_low, jnp.roll(x, -sp, axis=0), jnp.roll(x, sp, axis=0))
    hi = jnp.maximum(x, partner)
    lo = jnp.minimum(x, partner)
    return jnp.where(m_takemax, hi, lo)


def _transpose128(x):
    eye = jnp.eye(128, dtype=jnp.float32)
    rows, cols = x.shape
    if cols == 128:
        chunks = [
            jax.lax.dot_general(x[c * 128:(c + 1) * 128, :], eye,
                                (((0,), (0,)), ((), ())))
            for c in range(rows // 128)
        ]
        return jnp.concatenate(chunks, axis=1)
    chunks = [
        jax.lax.dot_general(x[:, c * 128:(c + 1) * 128], eye,
                            (((0,), (0,)), ((), ())))
        for c in range(cols // 128)
    ]
    return jnp.concatenate(chunks, axis=0)


def _topk_body(scores_ref, out_ref):
    xn = scores_ref[...]
    x = _transpose128(xn)
    iota = jax.lax.broadcasted_iota(jnp.int32, (N, 1), 0)

    for p in range(1, 10):
        for t in range(p - 1, -1, -1):
            sp = 8 << t
            if p <= 8:
                x = _stage_slice(x, sp, 8 << p)
            else:
                m = ((iota & sp) == 0) == ((iota & 1) == 0)
                x = _stage_mask(x, sp, m)

    for level in range(3):
        y = x.reshape(-1, 2, LANES)
        x = jnp.maximum(y[:, 0], y[:, 1])
        n = x.shape[0]
        k_il = n // 512
        it = iota[:n]
        for t in range(8, -1, -1):
            sp = k_il << t
            if level < 2:
                m = ((it & sp) == 0) == ((it & 1) == 0)
            else:
                m = (it & sp) == 0
            if sp >= 8:
                x = _stage_mask(x, sp, m)
            else:
                m_low = (it & sp) == 0
                x = _stage_roll(x, sp, m_low, m)

    out_ref[...] = _transpose128(x)


def _imp_body(q_ref, w_ref, b_ref, out_ref):
    z = jnp.sum(q_ref[...] * w_ref[...], axis=1, keepdims=True) + b_ref[0, 0]
    imp = jax.nn.sigmoid(z)
    out_ref[...] = (BASE_K + (MAX_K - BASE_K) * imp).astype(jnp.int32)


def kernel(Q, scores, W, b):
    rows = BATCH * SEQ
    scores_f = scores.reshape(rows, SEQ)
    q_f = Q.reshape(rows, DIM)

    vals = pl.pallas_call(
        _topk_body,
        grid=(rows // LANES,),
        in_specs=[pl.BlockSpec((LANES, SEQ), lambda i: (i, 0))],
        out_specs=pl.BlockSpec((LANES, MAX_K), lambda i: (i, 0)),
        out_shape=jax.ShapeDtypeStruct((rows, MAX_K), jnp.float32),
    )(scores_f)

    wb = W.reshape(1, DIM)
    bb = b.reshape(1, 1)
    k_out = pl.pallas_call(
        _imp_body,
        grid=(rows // 512,),
        in_specs=[
            pl.BlockSpec((512, DIM), lambda i: (i, 0)),
            pl.BlockSpec((1, DIM), lambda i: (0, 0)),
            pl.BlockSpec((1, 1), lambda i: (0, 0)),
        ],
        out_specs=pl.BlockSpec((512, 1), lambda i: (i, 0)),
        out_shape=jax.ShapeDtypeStruct((rows, 1), jnp.int32),
    )(q_f, wb, bb)

    return (vals.reshape(BATCH, SEQ, MAX_K), k_out.reshape(BATCH, SEQ))
